# baseline (speedup 1.0000x reference)
import functools

import jax
import jax.numpy as jnp
from jax.experimental import pallas as pl
from jax.experimental.pallas import tpu as pltpu

E = 20000
P = 320000
C_IN = 32
C_MID = 256
EMB = 32
LAT = 8
LONG = 8
MAXEL = 90
NB = 8
GROUPS = 4
SP = LAT * LONG

EB = 400


def _relu(v):
    return jnp.maximum(v, 0.0)


def _dense_block(a_ref, xd_ref, ohs_ref, oht_ref,
                 cw_ref, cb_ref, gnw_ref, gnb_ref,
                 d1w_ref, d1b_ref, d2w_ref, d2b_ref,
                 ms_ref, mt_ref, eb1_ref, eb2_ref,
                 f11w_ref, f11b_ref, f12w_ref, f12b_ref, f13w_ref, f13b_ref,
                 f21w_ref, f21b_ref, f22w_ref, f22b_ref, f23w_ref, f23b_ref,
                 out_ref):
    a = a_ref[...]

    parts = []
    for g in range(GROUPS):
        yg = jnp.dot(a[:, 512 * g:512 * (g + 1)], cw_ref[g],
                     preferred_element_type=jnp.float32)
        yg = _relu(yg + cb_ref[g][None, :])
        sg = yg[:, 0:64]
        for t in range(1, LONG):
            sg = sg + yg[:, 64 * t:64 * (t + 1)]
        parts.append(sg * (1.0 / LONG))

    normed = []
    for g in range(GROUPS):
        v = parts[g]
        mu = jnp.mean(v, axis=1, keepdims=True)
        c = v - mu
        var = jnp.mean(c * c, axis=1, keepdims=True)
        normed.append(c * jax.lax.rsqrt(var + 1e-5))
    h = jnp.concatenate(normed, axis=1)
    h = h * gnw_ref[...] + gnb_ref[...]

    logits_s = jnp.dot(ohs_ref[...], ms_ref[...],
                       preferred_element_type=jnp.float32)
    logits_t = jnp.dot(oht_ref[...], mt_ref[...],
                       preferred_element_type=jnp.float32)
    logits = logits_s + logits_t
    l1 = logits[:, 0:8] + eb1_ref[...]
    l2 = logits[:, 8:16] + eb2_ref[...]
    emb1 = jax.nn.softmax(l1, axis=1)
    emb2 = jax.nn.softmax(l2, axis=1)

    def embed(hin, emb, w1, b1, w2, b2, w3, b3):
        t1 = _relu(jnp.dot(hin, w1[...], preferred_element_type=jnp.float32)
                   + b1[...])
        t2 = _relu(jnp.dot(t1, w2[...], preferred_element_type=jnp.float32)
                   + b2[...])
        acc = t2[:, 0:256] * emb[:, 0:1]
        for nb in range(1, NB):
            acc = acc + t2[:, 256 * nb:256 * (nb + 1)] * emb[:, nb:nb + 1]
        return jnp.dot(acc, w3[...], preferred_element_type=jnp.float32) + b3[...]

    h = embed(h, emb1, f11w_ref, f11b_ref, f12w_ref, f12b_ref, f13w_ref, f13b_ref)

    xd = _relu(jnp.dot(xd_ref[...], d1w_ref[...],
                       preferred_element_type=jnp.float32) + d1b_ref[...])
    xd = jnp.dot(xd, d2w_ref[...], preferred_element_type=jnp.float32) + d2b_ref[...]

    h = _relu(h + xd)
    out_ref[...] = embed(h, emb2, f21w_ref, f21b_ref, f22w_ref, f22b_ref,
                         f23w_ref, f23b_ref)


def _dense_phase(A, x_dist, ohs, oht, CW, CB, gn_w, gn_b,
                 d1_w, d1_b, d2_w, d2_b, Ms, Mt, eb1, eb2,
                 e1w, e2w):
    nblk = E // EB
    full2 = lambda shape: pl.BlockSpec(shape, lambda i: (0, 0))
    full3 = lambda shape: pl.BlockSpec(shape, lambda i: (0, 0, 0))
    row = lambda w: pl.BlockSpec((EB, w), lambda i: (i, 0))

    in_specs = [
        row(2048), row(256), row(128), row(128),
        full3((GROUPS, 512, 512)), full2((GROUPS, 512)),
        full2((1, 256)), full2((1, 256)),
        full2((256, 256)), full2((1, 256)), full2((256, 256)), full2((1, 256)),
        full2((128, 16)), full2((128, 16)), full2((1, 8)), full2((1, 8)),
    ]
    for _ in range(2):
        in_specs += [full2((256, 256)), full2((1, 256)),
                     full2((256, 2048)), full2((1, 2048)),
                     full2((256, 256)), full2((1, 256))]

    return pl.pallas_call(
        _dense_block,
        grid=(nblk,),
        in_specs=in_specs,
        out_specs=pl.BlockSpec((EB, 256), lambda i: (i, 0)),
        out_shape=jax.ShapeDtypeStruct((E, 256), jnp.float32),
    )(A, x_dist, ohs, oht, CW, CB, gn_w.reshape(1, 256), gn_b.reshape(1, 256),
      d1_w.T, d1_b.reshape(1, 256), d2_w.T, d2_b.reshape(1, 256),
      Ms, Mt, eb1.reshape(1, 8), eb2.reshape(1, 8), *e1w, *e2w)


def kernel(x, x_dist, source_element, target_element, proj_index, proj_delta,
           proj_src_index, conv1_w, conv1_b, gn_w, gn_b, d1_w, d1_b, d2_w, d2_b,
           e1_src, e1_tgt, e1_efc_w, e1_efc_b, e1_fc1_w, e1_fc1_b, e1_fc2_w,
           e1_fc2_b, e1_fc3_w, e1_fc3_b, e2_src, e2_tgt, e2_efc_w, e2_efc_b,
           e2_fc1_w, e2_fc1_b, e2_fc2_w, e2_fc2_b, e2_fc3_w, e2_fc3_b):
    splat = x[proj_src_index]
    x_proj = jnp.zeros((E * SP, C_IN), x.dtype)
    for i in range(4):
        x_proj = x_proj.at[proj_index[i]].add(splat * proj_delta[i][:, None])
    A = x_proj.reshape(E, SP, GROUPS, 8).transpose(0, 2, 1, 3).reshape(E, 2048)

    W6 = conv1_w.reshape(GROUPS, 64, 8, 8, 8)
    jmap = (jnp.arange(8)[None, :] - jnp.arange(8)[:, None] + 4) % 8
    Wtm = W6[:, :, :, :, jmap]
    CW = Wtm.transpose(0, 3, 5, 2, 4, 1).reshape(GROUPS, 512, 512)
    CB = jnp.broadcast_to(conv1_b.reshape(GROUPS, 1, 64),
                          (GROUPS, 8, 64)).reshape(GROUPS, 512)

    M1s = e1_src @ e1_efc_w[:, :EMB].T
    M1t = e1_tgt @ e1_efc_w[:, EMB:].T
    M2s = e2_src @ e2_efc_w[:, :EMB].T
    M2t = e2_tgt @ e2_efc_w[:, EMB:].T
    Ms = jnp.pad(jnp.concatenate([M1s, M2s], axis=1), ((0, 128 - MAXEL), (0, 0)))
    Mt = jnp.pad(jnp.concatenate([M1t, M2t], axis=1), ((0, 128 - MAXEL), (0, 0)))

    el = jnp.arange(128, dtype=source_element.dtype)
    ohs = (source_element[:, None] == el[None, :]).astype(jnp.float32)
    oht = (target_element[:, None] == el[None, :]).astype(jnp.float32)

    e1w = (e1_fc1_w.T, e1_fc1_b.reshape(1, 256), e1_fc2_w.T,
           e1_fc2_b.reshape(1, 2048), e1_fc3_w.T, e1_fc3_b.reshape(1, 256))
    e2w = (e2_fc1_w.T, e2_fc1_b.reshape(1, 256), e2_fc2_w.T,
           e2_fc2_b.reshape(1, 2048), e2_fc3_w.T, e2_fc3_b.reshape(1, 256))

    return _dense_phase(A, x_dist, ohs, oht, CW, CB, gn_w, gn_b,
                        d1_w, d1_b, d2_w, d2_b, Ms, Mt, e1_efc_b, e2_efc_b,
                        e1w, e2w)


# speedup vs baseline: 2.0300x; 2.0300x over previous
import functools

import jax
import jax.numpy as jnp
from jax import lax
from jax.experimental import pallas as pl
from jax.experimental.pallas import tpu as pltpu
from jax.experimental.pallas import tpu_sc as plsc

E = 20000
P = 320000
C_IN = 32
C_MID = 256
EMB = 32
LAT = 8
LONG = 8
MAXEL = 90
NB = 8
GROUPS = 4
SP = LAT * LONG

EB = 400

NSC = 2
NTILE = 16
RCHUNK = 51200
NPASS = 13
TPR = RCHUNK // NTILE
PTS_PER_TILE = 20480
PPAD = PTS_PER_TILE * NTILE
KPTS = 512
NCHUNKS = PTS_PER_TILE // KPTS
CAP = 4 * KPTS
CAPPAD = 2176
ROWS_OUT = NSC * NPASS * RCHUNK
E_PAD = ROWS_OUT // SP
FB = 128
ZROWS = 200
WROWS = 200


def _sc_scatter_kernel(x_hbm, pts_hbm, out_hbm, acc, gsem):
    pl.run_scoped(
        functools.partial(_sc_scatter_body, x_hbm, pts_hbm, out_hbm, acc, gsem),
        pltpu.VMEM((9, KPTS), jnp.int32),
        pltpu.VMEM((CAPPAD,), jnp.int32),
        pltpu.VMEM((CAPPAD,), jnp.float32),
        pltpu.VMEM((CAPPAD // 128, 128), jnp.int32),
        pltpu.VMEM((4, 128), jnp.int32),
        pltpu.VMEM((FB, C_IN), jnp.float32),
        pltpu.VMEM((ZROWS, C_IN), jnp.float32),
    )


def _sc_scatter_body(x_hbm, pts_hbm, out_hbm, acc, gsem,
                     sbuf, cidx, cdel, csrc, bidx, gbuf, zbuf):
    core = lax.axis_index("c")
    sub = lax.axis_index("s")
    lane = lax.iota(jnp.int32, 16)
    zf = jnp.zeros((16,), jnp.float32)
    zi = jnp.zeros((16,), jnp.int32)

    def _init(v, _):
        r = v // 2
        c0 = (v % 2) * 16
        zbuf[r, pl.ds(c0, 16)] = zf
        return 0
    lax.fori_loop(0, ZROWS * 2, _init, 0)

    def _prefill(v, _):
        cidx[pl.ds(v * 16, 16)] = lane
        cdel[pl.ds(v * 16, 16)] = zf
        csrc[v // 8, pl.ds((v % 8) * 16, 16)] = lane
        return 0
    lax.fori_loop(0, CAPPAD // 16, _prefill, 0)

    def _flush(b, cursor):
        def _cp(v, _):
            bidx[0, pl.ds(v * 16, 16)] = cidx[pl.ds(b * FB + v * 16, 16)]
            return 0
        lax.fori_loop(0, 8, _cp, 0)
        pltpu.async_copy(x_hbm.at[csrc.at[b]], gbuf, gsem).wait()
        def _scale(j, _):
            jv = jnp.full((16,), b * FB + j, jnp.int32)
            d = plsc.load_gather(cdel, [jv])
            gbuf[j, pl.ds(0, 16)] = gbuf[j, pl.ds(0, 16)] * d
            gbuf[j, pl.ds(16, 16)] = gbuf[j, pl.ds(16, 16)] * d
            return 0
        lax.fori_loop(0, FB, _scale, 0)
        pltpu.sync_copy(gbuf, acc.at[bidx.at[0]], add=True)
        return cursor

    def _pass(p, _):
        chunk = p * NSC + core
        lo = chunk * RCHUNK
        for z in range(TPR // ZROWS):
            pltpu.sync_copy(zbuf, acc.at[pl.ds(sub * TPR + z * ZROWS, ZROWS), :])
        plsc.subcore_barrier()

        def _chunk(ci, _):
            start = sub * PTS_PER_TILE + ci * KPTS
            pltpu.sync_copy(pts_hbm.at[:, pl.ds(start, KPTS)], sbuf)
            cursor = jnp.int32(0)
            for i in range(4):
                def _scan(v, cur, i=i):
                    dest = sbuf[i, pl.ds(v * 16, 16)]
                    rel = dest - lo
                    m = (rel >= 0) & (rel < RCHUNK)
                    mi = m.astype(jnp.int32)
                    off = cur + jnp.cumsum(mi) - 1
                    plsc.store_scatter(cidx, [off], rel, mask=m)
                    dlt = plsc.bitcast(sbuf[4 + i, pl.ds(v * 16, 16)],
                                       jnp.float32)
                    plsc.store_scatter(cdel, [off], dlt, mask=m)
                    plsc.store_scatter(csrc, [off >> 7, off & 127],
                                       sbuf[8, pl.ds(v * 16, 16)], mask=m)
                    return cur + jnp.sum(mi)
                cursor = lax.fori_loop(0, KPTS // 16, _scan, cursor)
            nbat = (cursor + FB - 1) // FB
            lax.fori_loop(0, nbat, _flush, cursor)
            def _rst(v, _):
                cidx[pl.ds(v * 16, 16)] = lane
                cdel[pl.ds(v * 16, 16)] = zf
                csrc[v // 8, pl.ds((v % 8) * 16, 16)] = lane
                return 0
            lax.fori_loop(0, (cursor + 15) // 16, _rst, 0)
            return 0
        lax.fori_loop(0, NCHUNKS, _chunk, 0)
        plsc.subcore_barrier()
        rowbase = chunk * RCHUNK + sub * TPR
        nz = TPR // WROWS
        def _wb(t, _):
            g = t // nz
            z = t % nz
            pltpu.sync_copy(
                acc.at[pl.ds(sub * TPR + z * WROWS, WROWS), pl.ds(g * 8, 8)],
                out_hbm.at[g, pl.ds(rowbase + z * WROWS, WROWS), :])
            return 0
        lax.fori_loop(0, 4 * nz, _wb, 0)
        plsc.subcore_barrier()
        return 0
    lax.fori_loop(0, NPASS, _pass, 0)


def _sc_scatter(x, proj_index, proj_delta, proj_src_index):
    npad = PPAD - P
    padi = jnp.broadcast_to((jnp.arange(npad, dtype=proj_index.dtype)
                             % RCHUNK)[None, :], (4, npad))
    pidx = jnp.concatenate([proj_index, padi], axis=1)
    pdel = jnp.concatenate([proj_delta,
                            jnp.zeros((4, npad), proj_delta.dtype)], axis=1)
    psrc = jnp.concatenate([proj_src_index,
                            jnp.arange(npad, dtype=proj_src_index.dtype) % E])
    pts = jnp.concatenate([pidx.astype(jnp.int32),
                           lax.bitcast_convert_type(pdel, jnp.int32),
                           psrc.astype(jnp.int32)[None, :]], axis=0)
    mesh = plsc.VectorSubcoreMesh(core_axis_name="c", subcore_axis_name="s")
    f = pl.kernel(
        _sc_scatter_kernel,
        mesh=mesh,
        compiler_params=pltpu.CompilerParams(use_tc_tiling_on_sc=False, needs_layout_passes=False),
        out_type=jax.ShapeDtypeStruct((4, ROWS_OUT, 8), jnp.float32),
        scratch_types=[
            pltpu.VMEM_SHARED((RCHUNK, C_IN), jnp.float32),
            pltpu.SemaphoreType.DMA,
        ],
    )
    return f(x, pts)


def _relu(v):
    return jnp.maximum(v, 0.0)


def _dense_block(a0_ref, a1_ref, a2_ref, a3_ref, xd_ref, ohs_ref, oht_ref,
                 cw_ref, cb_ref, gnw_ref, gnb_ref,
                 d1w_ref, d1b_ref, d2w_ref, d2b_ref,
                 ms_ref, mt_ref, eb1_ref, eb2_ref,
                 f11w_ref, f11b_ref, f12w_ref, f12b_ref, f13w_ref, f13b_ref,
                 f21w_ref, f21b_ref, f22w_ref, f22b_ref, f23w_ref, f23b_ref,
                 out_ref):
    a_refs = (a0_ref, a1_ref, a2_ref, a3_ref)

    parts = []
    for g in range(GROUPS):
        yg = jnp.dot(a_refs[g][0], cw_ref[g],
                     preferred_element_type=jnp.float32)
        yg = _relu(yg + cb_ref[g][None, :])
        sg = yg[:, 0:64]
        for t in range(1, LONG):
            sg = sg + yg[:, 64 * t:64 * (t + 1)]
        parts.append(sg * (1.0 / LONG))

    normed = []
    for g in range(GROUPS):
        v = parts[g]
        mu = jnp.mean(v, axis=1, keepdims=True)
        c = v - mu
        var = jnp.mean(c * c, axis=1, keepdims=True)
        normed.append(c * jax.lax.rsqrt(var + 1e-5))
    h = jnp.concatenate(normed, axis=1)
    h = h * gnw_ref[...] + gnb_ref[...]

    logits_s = jnp.dot(ohs_ref[...], ms_ref[...],
                       preferred_element_type=jnp.float32)
    logits_t = jnp.dot(oht_ref[...], mt_ref[...],
                       preferred_element_type=jnp.float32)
    logits = logits_s + logits_t
    l1 = logits[:, 0:8] + eb1_ref[...]
    l2 = logits[:, 8:16] + eb2_ref[...]
    emb1 = jax.nn.softmax(l1, axis=1)
    emb2 = jax.nn.softmax(l2, axis=1)

    def embed(hin, emb, w1, b1, w2, b2, w3, b3):
        t1 = _relu(jnp.dot(hin, w1[...], preferred_element_type=jnp.float32)
                   + b1[...])
        t2 = _relu(jnp.dot(t1, w2[...], preferred_element_type=jnp.float32)
                   + b2[...])
        acc = t2[:, 0:256] * emb[:, 0:1]
        for nb in range(1, NB):
            acc = acc + t2[:, 256 * nb:256 * (nb + 1)] * emb[:, nb:nb + 1]
        return jnp.dot(acc, w3[...], preferred_element_type=jnp.float32) + b3[...]

    h = embed(h, emb1, f11w_ref, f11b_ref, f12w_ref, f12b_ref, f13w_ref, f13b_ref)

    xd = _relu(jnp.dot(xd_ref[...], d1w_ref[...],
                       preferred_element_type=jnp.float32) + d1b_ref[...])
    xd = jnp.dot(xd, d2w_ref[...], preferred_element_type=jnp.float32) + d2b_ref[...]

    h = _relu(h + xd)
    out_ref[...] = embed(h, emb2, f21w_ref, f21b_ref, f22w_ref, f22b_ref,
                         f23w_ref, f23b_ref)


def _dense_phase(A, x_dist, ohs, oht, CW, CB, gn_w, gn_b,
                 d1_w, d1_b, d2_w, d2_b, Ms, Mt, eb1, eb2,
                 e1w, e2w):
    nblk = E // EB
    full2 = lambda shape: pl.BlockSpec(shape, lambda i: (0, 0))
    full3 = lambda shape: pl.BlockSpec(shape, lambda i: (0, 0, 0))
    row = lambda w: pl.BlockSpec((EB, w), lambda i: (i, 0))
    agspec = lambda g: pl.BlockSpec((1, EB, 512), lambda i, g=g: (g, i, 0))

    in_specs = [
        agspec(0), agspec(1), agspec(2), agspec(3),
        row(256), row(128), row(128),
        full3((GROUPS, 512, 512)), full2((GROUPS, 512)),
        full2((1, 256)), full2((1, 256)),
        full2((256, 256)), full2((1, 256)), full2((256, 256)), full2((1, 256)),
        full2((128, 16)), full2((128, 16)), full2((1, 8)), full2((1, 8)),
    ]
    for _ in range(2):
        in_specs += [full2((256, 256)), full2((1, 256)),
                     full2((256, 2048)), full2((1, 2048)),
                     full2((256, 256)), full2((1, 256))]

    return pl.pallas_call(
        _dense_block,
        grid=(nblk,),
        in_specs=in_specs,
        out_specs=pl.BlockSpec((EB, 256), lambda i: (i, 0)),
        out_shape=jax.ShapeDtypeStruct((E, 256), jnp.float32),
    )(A, A, A, A, x_dist, ohs, oht, CW, CB, gn_w.reshape(1, 256), gn_b.reshape(1, 256),
      d1_w.T, d1_b.reshape(1, 256), d2_w.T, d2_b.reshape(1, 256),
      Ms, Mt, eb1.reshape(1, 8), eb2.reshape(1, 8), *e1w, *e2w)


def kernel(x, x_dist, source_element, target_element, proj_index, proj_delta,
           proj_src_index, conv1_w, conv1_b, gn_w, gn_b, d1_w, d1_b, d2_w, d2_b,
           e1_src, e1_tgt, e1_efc_w, e1_efc_b, e1_fc1_w, e1_fc1_b, e1_fc2_w,
           e1_fc2_b, e1_fc3_w, e1_fc3_b, e2_src, e2_tgt, e2_efc_w, e2_efc_b,
           e2_fc1_w, e2_fc1_b, e2_fc2_w, e2_fc2_b, e2_fc3_w, e2_fc3_b):
    A = _sc_scatter(x, proj_index, proj_delta,
                    proj_src_index).reshape(GROUPS, E_PAD, 512)

    W6 = conv1_w.reshape(GROUPS, 64, 8, 8, 8)
    jmap = (jnp.arange(8)[None, :] - jnp.arange(8)[:, None] + 4) % 8
    Wtm = W6[:, :, :, :, jmap]
    CW = Wtm.transpose(0, 3, 5, 2, 4, 1).reshape(GROUPS, 512, 512)
    CB = jnp.broadcast_to(conv1_b.reshape(GROUPS, 1, 64),
                          (GROUPS, 8, 64)).reshape(GROUPS, 512)

    M1s = e1_src @ e1_efc_w[:, :EMB].T
    M1t = e1_tgt @ e1_efc_w[:, EMB:].T
    M2s = e2_src @ e2_efc_w[:, :EMB].T
    M2t = e2_tgt @ e2_efc_w[:, EMB:].T
    Ms = jnp.pad(jnp.concatenate([M1s, M2s], axis=1), ((0, 128 - MAXEL), (0, 0)))
    Mt = jnp.pad(jnp.concatenate([M1t, M2t], axis=1), ((0, 128 - MAXEL), (0, 0)))

    el = jnp.arange(128, dtype=source_element.dtype)
    ohs = (source_element[:, None] == el[None, :]).astype(jnp.float32)
    oht = (target_element[:, None] == el[None, :]).astype(jnp.float32)

    e1w = (e1_fc1_w.T, e1_fc1_b.reshape(1, 256), e1_fc2_w.T,
           e1_fc2_b.reshape(1, 2048), e1_fc3_w.T, e1_fc3_b.reshape(1, 256))
    e2w = (e2_fc1_w.T, e2_fc1_b.reshape(1, 256), e2_fc2_w.T,
           e2_fc2_b.reshape(1, 2048), e2_fc3_w.T, e2_fc3_b.reshape(1, 256))

    return _dense_phase(A, x_dist, ohs, oht, CW, CB, gn_w, gn_b,
                        d1_w, d1_b, d2_w, d2_b, Ms, Mt, e1_efc_b, e2_efc_b,
                        e1w, e2w)
